# Initial kernel scaffold; baseline (speedup 1.0000x reference)
import jax
import jax.numpy as jnp
from jax.experimental import pallas as pl


def kernel(input_ids, token_table, pos_table):
    raise NotImplementedError("write your pallas kernel here")


# speedup vs baseline: 1.4572x; 1.4572x over previous
import functools

import jax
import jax.numpy as jnp
from jax import lax
from jax.experimental import pallas as pl
from jax.experimental.pallas import tpu as pltpu
from jax.experimental.pallas import tpu_sc as plsc

VOCAB = 100000
D = 1024
SEQ = 4096
B = 4

NC = 2
NS = 16
NW = NC * NS

SPAN = SEQ // NW
C = 16
NCHUNK = SPAN // C
LANES = 16
GROUPS = D // LANES


def _body(ids_hbm, tok_hbm, pos_hbm, out_hbm, idx_v, tok_v, pos_v, sem):
    cid = lax.axis_index("c")
    sid = lax.axis_index("s")
    wid = sid * NC + cid
    seq0 = wid * SPAN

    for ck in range(NCHUNK):
        s0 = seq0 + ck * C
        for b in range(B):
            pltpu.sync_copy(ids_hbm.at[b, pl.ds(s0, C)], idx_v.at[b])
        cp_pos = pltpu.async_copy(pos_hbm.at[pl.ds(s0, C)], pos_v, sem)
        cps = [
            pltpu.async_copy(tok_hbm.at[idx_v.at[b]], tok_v.at[b], sem)
            for b in range(B)
        ]
        cp_pos.wait()
        for cp in cps:
            cp.wait()

        def add_body(g, _):
            r = g // GROUPS
            j = (g % GROUPS) * LANES
            pvec = pos_v[r, pl.ds(j, LANES)]
            for b in range(B):
                plsc.addupdate(tok_v.at[b, r, pl.ds(j, LANES)], pvec)
            return _

        lax.fori_loop(0, C * GROUPS, add_body, 0)

        for b in range(B):
            pltpu.sync_copy(tok_v.at[b], out_hbm.at[b, pl.ds(s0, C)])


@functools.partial(jax.jit, static_argnames=())
def kernel(input_ids, token_table, pos_table):
    ids = input_ids.astype(jnp.int32)
    mesh = plsc.VectorSubcoreMesh(core_axis_name="c", subcore_axis_name="s")
    k = pl.kernel(
        _body,
        out_type=jax.ShapeDtypeStruct((B, SEQ, D), jnp.float32),
        mesh=mesh,
        scratch_types=[
            pltpu.VMEM((B, C), jnp.int32),
            pltpu.VMEM((B, C, D), jnp.float32),
            pltpu.VMEM((C, D), jnp.float32),
            pltpu.SemaphoreType.DMA,
        ],
    )
    return k(ids, token_table, pos_table)


# speedup vs baseline: 1.9349x; 1.3278x over previous
import functools

import jax
import jax.numpy as jnp
from jax import lax
from jax.experimental import pallas as pl
from jax.experimental.pallas import tpu as pltpu
from jax.experimental.pallas import tpu_sc as plsc

VOCAB = 100000
D = 1024
SEQ = 4096
B = 4

NC = 2
NS = 16
NW = NC * NS

SPAN = SEQ // NW
C = 8
NCHUNK = SPAN // C
NBUF = 2
LANES = 16
GROUPS = D // LANES


def _body(ids_hbm, tok_hbm, pos_hbm, out_hbm, idx_v, tok_v, pos_v,
          in_sem0, in_sem1, out_sem0, out_sem1):
    cid = lax.axis_index("c")
    sid = lax.axis_index("s")
    wid = sid * NC + cid
    seq0 = wid * SPAN

    in_sems = [in_sem0, in_sem1]
    out_sems = [out_sem0, out_sem1]

    def issue_in(ck, p):
        s0 = seq0 + ck * C
        for b in range(B):
            pltpu.sync_copy(ids_hbm.at[b, pl.ds(s0, C)], idx_v.at[p, b])
        descs = [pltpu.async_copy(pos_hbm.at[pl.ds(s0, C)], pos_v.at[p],
                                  in_sems[p])]
        descs += [
            pltpu.async_copy(tok_hbm.at[idx_v.at[p, b]], tok_v.at[p, b],
                             in_sems[p])
            for b in range(B)
        ]
        return descs

    in_descs = [None] * NBUF
    out_descs = [None] * NBUF
    in_descs[0] = issue_in(0, 0)

    for ck in range(NCHUNK):
        p = ck % NBUF
        nxt = ck + 1
        if nxt < NCHUNK:
            q = nxt % NBUF
            if out_descs[q] is not None:
                for d_ in out_descs[q]:
                    d_.wait()
                out_descs[q] = None
            in_descs[q] = issue_in(nxt, q)

        for d_ in in_descs[p]:
            d_.wait()

        @plsc.parallel_loop(0, C * GROUPS, step=1, unroll=8)
        def add_body(g):
            r = g // GROUPS
            off = (g % GROUPS) * LANES
            pvec = pos_v[p, r, pl.ds(off, LANES)]
            for b in range(B):
                plsc.addupdate(tok_v.at[p, b, r, pl.ds(off, LANES)], pvec)

        s0 = seq0 + ck * C
        out_descs[p] = [
            pltpu.async_copy(tok_v.at[p, b], out_hbm.at[b, pl.ds(s0, C)],
                             out_sems[p])
            for b in range(B)
        ]

    for ds_ in out_descs:
        if ds_ is not None:
            for d_ in ds_:
                d_.wait()


@jax.jit
def kernel(input_ids, token_table, pos_table):
    ids = input_ids.astype(jnp.int32)
    mesh = plsc.VectorSubcoreMesh(core_axis_name="c", subcore_axis_name="s")
    k = pl.kernel(
        _body,
        out_type=jax.ShapeDtypeStruct((B, SEQ, D), jnp.float32),
        mesh=mesh,
        scratch_types=[
            pltpu.VMEM((NBUF, B, C), jnp.int32),
            pltpu.VMEM((NBUF, B, C, D), jnp.float32),
            pltpu.VMEM((NBUF, C, D), jnp.float32),
            pltpu.SemaphoreType.DMA,
            pltpu.SemaphoreType.DMA,
            pltpu.SemaphoreType.DMA,
            pltpu.SemaphoreType.DMA,
        ],
    )
    return k(ids, token_table, pos_table)


# speedup vs baseline: 2.3818x; 1.2310x over previous
import functools

import jax
import jax.numpy as jnp
from jax import lax
from jax.experimental import pallas as pl
from jax.experimental.pallas import tpu as pltpu
from jax.experimental.pallas import tpu_sc as plsc

VOCAB = 100000
D = 1024
SEQ = 4096
B = 4

NC = 2
NS = 16
NW = NC * NS

SPAN = SEQ // NW
C = 8
NCHUNK = SPAN // C
NBUF = 3
LANES = 16
GROUPS = D // LANES


def _body(ids_hbm, tok_hbm, pos_hbm, out_hbm, idx_v, tok_v, pos_v,
          in_sem0, in_sem1, in_sem2, out_sem0, out_sem1, out_sem2):
    cid = lax.axis_index("c")
    sid = lax.axis_index("s")
    wid = sid * NC + cid
    seq0 = wid * SPAN

    in_sems = [in_sem0, in_sem1, in_sem2]
    out_sems = [out_sem0, out_sem1, out_sem2]

    for b in range(B):
        pltpu.sync_copy(ids_hbm.at[b, pl.ds(seq0, SPAN)], idx_v.at[b])

    def issue_in(ck, p):
        s0 = seq0 + ck * C
        descs = [pltpu.async_copy(pos_hbm.at[pl.ds(s0, C)], pos_v.at[p],
                                  in_sems[p])]
        descs += [
            pltpu.async_copy(tok_hbm.at[idx_v.at[b, pl.ds(ck * C, C)]],
                             tok_v.at[p, b], in_sems[p])
            for b in range(B)
        ]
        return descs

    in_descs = [None] * NBUF
    out_descs = [None] * NBUF
    for w in range(NBUF - 1):
        in_descs[w] = issue_in(w, w)

    for ck in range(NCHUNK):
        p = ck % NBUF
        nxt = ck + NBUF - 1
        if nxt < NCHUNK:
            q = nxt % NBUF
            if out_descs[q] is not None:
                for d_ in out_descs[q]:
                    d_.wait()
                out_descs[q] = None
            in_descs[q] = issue_in(nxt, q)

        for d_ in in_descs[p]:
            d_.wait()

        @plsc.parallel_loop(0, C * GROUPS, step=1, unroll=8)
        def add_body(g):
            r = g // GROUPS
            off = (g % GROUPS) * LANES
            pvec = pos_v[p, r, pl.ds(off, LANES)]
            for b in range(B):
                plsc.addupdate(tok_v.at[p, b, r, pl.ds(off, LANES)], pvec)

        s0 = seq0 + ck * C
        out_descs[p] = [
            pltpu.async_copy(tok_v.at[p, b], out_hbm.at[b, pl.ds(s0, C)],
                             out_sems[p])
            for b in range(B)
        ]

    for ds_ in out_descs:
        if ds_ is not None:
            for d_ in ds_:
                d_.wait()


@jax.jit
def kernel(input_ids, token_table, pos_table):
    ids = input_ids.astype(jnp.int32)
    mesh = plsc.VectorSubcoreMesh(core_axis_name="c", subcore_axis_name="s")
    k = pl.kernel(
        _body,
        out_type=jax.ShapeDtypeStruct((B, SEQ, D), jnp.float32),
        mesh=mesh,
        scratch_types=[
            pltpu.VMEM((B, SPAN), jnp.int32),
            pltpu.VMEM((NBUF, B, C, D), jnp.float32),
            pltpu.VMEM((NBUF, C, D), jnp.float32),
            pltpu.SemaphoreType.DMA,
            pltpu.SemaphoreType.DMA,
            pltpu.SemaphoreType.DMA,
            pltpu.SemaphoreType.DMA,
            pltpu.SemaphoreType.DMA,
            pltpu.SemaphoreType.DMA,
        ],
    )
    return k(ids, token_table, pos_table)


# speedup vs baseline: 2.4353x; 1.0225x over previous
import jax
import jax.numpy as jnp
from jax import lax
from jax.experimental import pallas as pl
from jax.experimental.pallas import tpu as pltpu
from jax.experimental.pallas import tpu_sc as plsc

VOCAB = 100000
D = 1024
SEQ = 4096
B = 4

NC = 2
NS = 16
NW = NC * NS

SPAN = SEQ // NW
C = 8
NCHUNK = SPAN // C
NBUF = 3
ROWS = B * C
LANES = 16
GROUPS = D // LANES


def _body(ids_hbm, tok_hbm, pos_hbm, out_hbm, idx_v, tok_v, pos_v,
          in_sem0, in_sem1, in_sem2, out_sem0, out_sem1, out_sem2):
    cid = lax.axis_index("c")
    sid = lax.axis_index("s")
    wid = sid * NC + cid
    seq0 = wid * SPAN

    in_sems = [in_sem0, in_sem1, in_sem2]
    out_sems = [out_sem0, out_sem1, out_sem2]

    pltpu.sync_copy(ids_hbm.at[wid], idx_v)

    def issue_in(ck, p):
        s0 = seq0 + ck * C
        return [
            pltpu.async_copy(pos_hbm.at[pl.ds(s0, C)], pos_v.at[p],
                             in_sems[p]),
            pltpu.async_copy(tok_hbm.at[idx_v.at[ck]], tok_v.at[p],
                             in_sems[p]),
        ]

    in_descs = [None] * NBUF
    out_descs = [None] * NBUF
    for w in range(NBUF - 1):
        in_descs[w] = issue_in(w, w)

    for ck in range(NCHUNK):
        p = ck % NBUF
        nxt = ck + NBUF - 1
        if nxt < NCHUNK:
            q = nxt % NBUF
            if out_descs[q] is not None:
                for d_ in out_descs[q]:
                    d_.wait()
                out_descs[q] = None
            in_descs[q] = issue_in(nxt, q)

        for d_ in in_descs[p]:
            d_.wait()

        @plsc.parallel_loop(0, C * GROUPS, step=1, unroll=8)
        def add_body(g):
            r = g // GROUPS
            off = (g % GROUPS) * LANES
            pvec = pos_v[p, r, pl.ds(off, LANES)]
            for b in range(B):
                plsc.addupdate(tok_v.at[p, b * C + r, pl.ds(off, LANES)],
                               pvec)

        s0 = seq0 + ck * C
        out_descs[p] = [
            pltpu.async_copy(tok_v.at[p, pl.ds(b * C, C)],
                             out_hbm.at[b, pl.ds(s0, C)], out_sems[p])
            for b in range(B)
        ]

    for ds_ in out_descs:
        if ds_ is not None:
            for d_ in ds_:
                d_.wait()


@jax.jit
def kernel(input_ids, token_table, pos_table):
    ids = input_ids.astype(jnp.int32)
    ids_r = (
        ids.reshape(B, NW, NCHUNK, C)
        .transpose(1, 2, 0, 3)
        .reshape(NW, NCHUNK, ROWS)
    )
    mesh = plsc.VectorSubcoreMesh(core_axis_name="c", subcore_axis_name="s")
    k = pl.kernel(
        _body,
        out_type=jax.ShapeDtypeStruct((B, SEQ, D), jnp.float32),
        mesh=mesh,
        scratch_types=[
            pltpu.VMEM((NCHUNK, ROWS), jnp.int32),
            pltpu.VMEM((NBUF, ROWS, D), jnp.float32),
            pltpu.VMEM((NBUF, C, D), jnp.float32),
            pltpu.SemaphoreType.DMA,
            pltpu.SemaphoreType.DMA,
            pltpu.SemaphoreType.DMA,
            pltpu.SemaphoreType.DMA,
            pltpu.SemaphoreType.DMA,
            pltpu.SemaphoreType.DMA,
        ],
    )
    return k(ids_r, token_table, pos_table)
